# baseline (speedup 1.0000x reference)
import functools

import jax
import jax.numpy as jnp
from jax import lax
from jax.experimental import pallas as pl
from jax.experimental.pallas import tpu as pltpu
from jax.experimental.pallas import tpu_sc as plsc

N_CLS_K = 1000
SEQ_K = 77
N_CTX_K = 16
CTX_DIM_K = 512
CHUNK = 48
N_CHUNKS = 21
LAST_START = N_CLS_K - CHUNK
N_TASKS = SEQ_K * N_CHUNKS
NBUF = 4

NC, NS = 2, 16
NW = NC * NS
BASE_T = N_TASKS // NW
EXTRA_T = N_TASKS - BASE_T * NW
ROUNDS = 52
assert ROUNDS % NBUF == 0 and ROUNDS >= BASE_T + 1


def _make_kernel():
    mesh = plsc.VectorSubcoreMesh(core_axis_name="c", subcore_axis_name="s")

    @functools.partial(
        pl.kernel,
        mesh=mesh,
        out_type=jax.ShapeDtypeStruct((SEQ_K, N_CLS_K, CTX_DIM_K), jnp.float32),
        scratch_types=(
            [pltpu.VMEM((CHUNK, CTX_DIM_K), jnp.float32) for _ in range(NBUF)]
            + [pltpu.VMEM((CHUNK,), jnp.int32) for _ in range(NBUF)]
            + [pltpu.SemaphoreType.DMA for _ in range(3 * NBUF)]
        ),
    )
    def sc_kernel(idx_hbm, table_hbm, ctxb_hbm, out_hbm, *scratch):
        bufs = scratch[:NBUF]
        ibufs = scratch[NBUF : 2 * NBUF]
        sidx = scratch[2 * NBUF : 3 * NBUF]
        sin = scratch[3 * NBUF : 4 * NBUF]
        sw = scratch[4 * NBUF : 5 * NBUF]
        wid = lax.axis_index("s") * NC + lax.axis_index("c")
        n_t = jnp.where(wid < EXTRA_T, BASE_T + 1, BASE_T)

        def params(t_local):
            t = wid + NW * lax.rem(t_local, n_t)
            s = t // N_CHUNKS
            ch = lax.rem(t, N_CHUNKS)
            c0 = pl.multiple_of(
                jnp.where(ch == N_CHUNKS - 1, LAST_START, ch * CHUNK), 8
            )
            is_ctx = jnp.logical_and(s >= 1, s < 1 + N_CTX_K)
            row = jnp.where(jnp.logical_or(is_ctx, s == 0), 0, s - N_CTX_K)
            ioff = pl.multiple_of(row * N_CLS_K + c0, 8)
            return s, c0, is_ctx, ioff

        def stage_idx(t_local, b):
            _, _, _, ioff = params(t_local)
            pltpu.async_copy(idx_hbm.at[pl.ds(ioff, CHUNK)], ibufs[b], sidx[b])

        def wait_idx(b):
            pltpu.make_async_copy(
                idx_hbm.at[pl.ds(0, CHUNK)], ibufs[b], sidx[b]
            ).wait()

        def issue_main(t_local, b):
            s, _, is_ctx, _ = params(t_local)

            @pl.when(is_ctx)
            def _():
                pltpu.async_copy(
                    ctxb_hbm.at[s - 1], bufs[b].at[pl.ds(0, 16)], sin[b]
                )

            @pl.when(jnp.logical_not(is_ctx))
            def _():
                pltpu.async_copy(table_hbm.at[ibufs[b]], bufs[b], sin[b])

        def wait_main(t_local, b):
            _, _, is_ctx, _ = params(t_local)

            @pl.when(is_ctx)
            def _():
                pltpu.make_async_copy(
                    ctxb_hbm.at[0], bufs[b].at[pl.ds(0, 16)], sin[b]
                ).wait()

            @pl.when(jnp.logical_not(is_ctx))
            def _():
                pltpu.make_async_copy(
                    table_hbm.at[pl.ds(0, CHUNK)], bufs[b], sin[b]
                ).wait()

        def issue_write(t_local, b):
            s, c0, is_ctx, _ = params(t_local)

            @pl.when(is_ctx)
            def _():
                for i in range(CHUNK // 16):
                    pltpu.async_copy(
                        bufs[b].at[pl.ds(0, 16)],
                        out_hbm.at[s, pl.ds(c0 + 16 * i, 16)],
                        sw[b],
                    )

            @pl.when(jnp.logical_not(is_ctx))
            def _():
                pltpu.async_copy(
                    bufs[b], out_hbm.at[s, pl.ds(c0, CHUNK)], sw[b]
                )

        def drain_write(b):
            pltpu.make_async_copy(
                bufs[b], out_hbm.at[0, pl.ds(0, CHUNK)], sw[b]
            ).wait()

        stage_idx(0, 0)
        stage_idx(1, 1)
        stage_idx(2, 2)
        wait_idx(0)
        issue_main(0, 0)
        wait_idx(1)
        issue_main(1, 1)

        def quad(tt, carry):
            for b in range(NBUF):
                t = tt * NBUF + b
                b2, b3 = (b + 2) % NBUF, (b + 3) % NBUF

                @pl.when(t < n_t)
                def _():
                    wait_main(t, b)
                    issue_write(t, b)

                @pl.when(t + 3 < n_t)
                def _():
                    stage_idx(t + 3, b3)

                @pl.when(t + 2 < n_t)
                def _():
                    wait_idx(b2)

                    @pl.when(t >= 2)
                    def _():
                        drain_write(b2)

                    issue_main(t + 2, b2)
            return carry

        lax.fori_loop(0, ROUNDS // NBUF, quad, 0)
        for b in range(NBUF):
            drain_write(b)

    return sc_kernel


_SC_KERNEL = _make_kernel()


@jax.jit
def kernel(tokenized_prompts, token_embedding, ctx):
    cols = jnp.concatenate(
        [tokenized_prompts[:, :1], tokenized_prompts[:, 1 + N_CTX_K :]], axis=1
    )
    idx = cols.T.reshape(-1)
    ctxb = jnp.broadcast_to(ctx[:, None, :], (N_CTX_K, 16, CTX_DIM_K))
    out_t = _SC_KERNEL(idx, token_embedding, ctxb)
    return jnp.transpose(out_t, (1, 0, 2))


# speedup vs baseline: 1.0274x; 1.0274x over previous
import functools

import jax
import jax.numpy as jnp
from jax import lax
from jax.experimental import pallas as pl
from jax.experimental.pallas import tpu as pltpu
from jax.experimental.pallas import tpu_sc as plsc

N_CLS_K = 1000
SEQ_K = 77
N_CTX_K = 16
CTX_DIM_K = 512
CHUNK = 48
N_CHUNKS = 21
LAST_START = N_CLS_K - CHUNK
N_TASKS = SEQ_K * N_CHUNKS
NBUF = 4

NC, NS = 2, 16
NW = NC * NS
BASE_T = N_TASKS // NW
EXTRA_T = N_TASKS - BASE_T * NW
ROUNDS = 52
assert ROUNDS % NBUF == 0 and ROUNDS >= BASE_T + 1


def _make_kernel():
    mesh = plsc.VectorSubcoreMesh(core_axis_name="c", subcore_axis_name="s")

    @functools.partial(
        pl.kernel,
        mesh=mesh,
        out_type=jax.ShapeDtypeStruct((SEQ_K, N_CLS_K, CTX_DIM_K), jnp.float32),
        scratch_types=(
            [pltpu.VMEM((CHUNK, CTX_DIM_K), jnp.float32) for _ in range(NBUF)]
            + [pltpu.VMEM((CHUNK,), jnp.int32) for _ in range(NBUF)]
            + [pltpu.SemaphoreType.DMA for _ in range(3 * NBUF)]
        ),
    )
    def sc_kernel(idx_hbm, table_hbm, ctxb_hbm, out_hbm, *scratch):
        bufs = scratch[:NBUF]
        ibufs = scratch[NBUF : 2 * NBUF]
        sidx = scratch[2 * NBUF : 3 * NBUF]
        sin = scratch[3 * NBUF : 4 * NBUF]
        sw = scratch[4 * NBUF : 5 * NBUF]
        wid = lax.axis_index("s") * NC + lax.axis_index("c")
        n_t = jnp.where(wid < EXTRA_T, BASE_T + 1, BASE_T)

        def params(t_local):
            t = wid + NW * lax.rem(t_local, n_t)
            s = t // N_CHUNKS
            ch = lax.rem(t, N_CHUNKS)
            c0 = pl.multiple_of(
                jnp.where(ch == N_CHUNKS - 1, LAST_START, ch * CHUNK), 8
            )
            is_ctx = jnp.logical_and(s >= 1, s < 1 + N_CTX_K)
            row = jnp.where(jnp.logical_or(is_ctx, s == 0), 0, s - N_CTX_K)
            ioff = pl.multiple_of(row * N_CLS_K + c0, 8)
            return s, c0, is_ctx, ioff

        def stage_idx(t_local, b):
            _, _, _, ioff = params(t_local)
            pltpu.async_copy(idx_hbm.at[pl.ds(ioff, CHUNK)], ibufs[b], sidx[b])

        def wait_idx(b):
            pltpu.make_async_copy(
                idx_hbm.at[pl.ds(0, CHUNK)], ibufs[b], sidx[b]
            ).wait()

        def issue_main(t_local, b):
            s, _, is_ctx, _ = params(t_local)

            @pl.when(is_ctx)
            def _():
                pltpu.async_copy(
                    ctxb_hbm.at[s - 1], bufs[b].at[pl.ds(0, 8)], sin[b]
                )

            @pl.when(jnp.logical_not(is_ctx))
            def _():
                pltpu.async_copy(table_hbm.at[ibufs[b]], bufs[b], sin[b])

        def wait_main(t_local, b):
            _, _, is_ctx, _ = params(t_local)

            @pl.when(is_ctx)
            def _():
                pltpu.make_async_copy(
                    ctxb_hbm.at[0], bufs[b].at[pl.ds(0, 8)], sin[b]
                ).wait()

            @pl.when(jnp.logical_not(is_ctx))
            def _():
                pltpu.make_async_copy(
                    table_hbm.at[pl.ds(0, CHUNK)], bufs[b], sin[b]
                ).wait()

        def issue_write(t_local, b):
            s, c0, is_ctx, _ = params(t_local)

            @pl.when(is_ctx)
            def _():
                for i in range(CHUNK // 8):
                    pltpu.async_copy(
                        bufs[b].at[pl.ds(0, 8)],
                        out_hbm.at[s, pl.ds(c0 + 8 * i, 8)],
                        sw[b],
                    )

            @pl.when(jnp.logical_not(is_ctx))
            def _():
                pltpu.async_copy(
                    bufs[b], out_hbm.at[s, pl.ds(c0, CHUNK)], sw[b]
                )

        def drain_write(b):
            pltpu.make_async_copy(
                bufs[b], out_hbm.at[0, pl.ds(0, CHUNK)], sw[b]
            ).wait()

        stage_idx(0, 0)
        stage_idx(1, 1)
        stage_idx(2, 2)
        wait_idx(0)
        issue_main(0, 0)
        wait_idx(1)
        issue_main(1, 1)

        def quad(tt, carry):
            for b in range(NBUF):
                t = tt * NBUF + b
                b2, b3 = (b + 2) % NBUF, (b + 3) % NBUF

                @pl.when(t < n_t)
                def _():
                    wait_main(t, b)
                    issue_write(t, b)

                @pl.when(t + 3 < n_t)
                def _():
                    stage_idx(t + 3, b3)

                @pl.when(t + 2 < n_t)
                def _():
                    wait_idx(b2)

                    @pl.when(t >= 2)
                    def _():
                        drain_write(b2)

                    issue_main(t + 2, b2)
            return carry

        lax.fori_loop(0, ROUNDS // NBUF, quad, 0)
        for b in range(NBUF):
            drain_write(b)

    return sc_kernel


_SC_KERNEL = _make_kernel()


@jax.jit
def kernel(tokenized_prompts, token_embedding, ctx):
    cols = jnp.concatenate(
        [tokenized_prompts[:, :1], tokenized_prompts[:, 1 + N_CTX_K :]], axis=1
    )
    idx = cols.T.reshape(-1)
    ctxb = jnp.broadcast_to(ctx[:, None, :], (N_CTX_K, 8, CTX_DIM_K))
    out_t = _SC_KERNEL(idx, token_embedding, ctxb)
    return jnp.transpose(out_t, (1, 0, 2))


# speedup vs baseline: 1.0348x; 1.0072x over previous
import functools

import jax
import jax.numpy as jnp
from jax import lax
from jax.experimental import pallas as pl
from jax.experimental.pallas import tpu as pltpu
from jax.experimental.pallas import tpu_sc as plsc

N_CLS_K = 1000
SEQ_K = 77
N_CTX_K = 16
CTX_DIM_K = 512
CHUNK = 48
N_CHUNKS = 21
LAST_START = N_CLS_K - CHUNK
N_TASKS = SEQ_K * N_CHUNKS
NBUF = 5

NC, NS = 2, 16
NW = NC * NS
BASE_T = N_TASKS // NW
EXTRA_T = N_TASKS - BASE_T * NW
ROUNDS = 55
assert ROUNDS % NBUF == 0 and ROUNDS >= BASE_T + 1


def _make_kernel():
    mesh = plsc.VectorSubcoreMesh(core_axis_name="c", subcore_axis_name="s")

    @functools.partial(
        pl.kernel,
        mesh=mesh,
        out_type=jax.ShapeDtypeStruct((SEQ_K, N_CLS_K, CTX_DIM_K), jnp.float32),
        scratch_types=(
            [pltpu.VMEM((CHUNK, CTX_DIM_K), jnp.float32) for _ in range(NBUF)]
            + [pltpu.VMEM((CHUNK,), jnp.int32) for _ in range(NBUF)]
            + [pltpu.SemaphoreType.DMA for _ in range(3 * NBUF)]
        ),
    )
    def sc_kernel(idx_hbm, table_hbm, ctxb_hbm, out_hbm, *scratch):
        bufs = scratch[:NBUF]
        ibufs = scratch[NBUF : 2 * NBUF]
        sidx = scratch[2 * NBUF : 3 * NBUF]
        sin = scratch[3 * NBUF : 4 * NBUF]
        sw = scratch[4 * NBUF : 5 * NBUF]
        wid = lax.axis_index("s") * NC + lax.axis_index("c")
        n_t = jnp.where(wid < EXTRA_T, BASE_T + 1, BASE_T)

        def params(t_local):
            t = wid + NW * lax.rem(t_local, n_t)
            s = t // N_CHUNKS
            ch = lax.rem(t, N_CHUNKS)
            c0 = pl.multiple_of(
                jnp.where(ch == N_CHUNKS - 1, LAST_START, ch * CHUNK), 8
            )
            is_ctx = jnp.logical_and(s >= 1, s < 1 + N_CTX_K)
            row = jnp.where(jnp.logical_or(is_ctx, s == 0), 0, s - N_CTX_K)
            ioff = pl.multiple_of(row * N_CLS_K + c0, 8)
            return s, c0, is_ctx, ioff

        def stage_idx(t_local, b):
            _, _, _, ioff = params(t_local)
            pltpu.async_copy(idx_hbm.at[pl.ds(ioff, CHUNK)], ibufs[b], sidx[b])

        def wait_idx(b):
            pltpu.make_async_copy(
                idx_hbm.at[pl.ds(0, CHUNK)], ibufs[b], sidx[b]
            ).wait()

        def issue_main(t_local, b):
            s, _, is_ctx, _ = params(t_local)

            @pl.when(is_ctx)
            def _():
                pltpu.async_copy(
                    ctxb_hbm.at[s - 1], bufs[b].at[pl.ds(0, 8)], sin[b]
                )

            @pl.when(jnp.logical_not(is_ctx))
            def _():
                pltpu.async_copy(table_hbm.at[ibufs[b]], bufs[b], sin[b])

        def wait_main(t_local, b):
            _, _, is_ctx, _ = params(t_local)

            @pl.when(is_ctx)
            def _():
                pltpu.make_async_copy(
                    ctxb_hbm.at[0], bufs[b].at[pl.ds(0, 8)], sin[b]
                ).wait()

            @pl.when(jnp.logical_not(is_ctx))
            def _():
                pltpu.make_async_copy(
                    table_hbm.at[pl.ds(0, CHUNK)], bufs[b], sin[b]
                ).wait()

        def issue_write(t_local, b):
            s, c0, is_ctx, _ = params(t_local)

            @pl.when(is_ctx)
            def _():
                for i in range(CHUNK // 8):
                    pltpu.async_copy(
                        bufs[b].at[pl.ds(0, 8)],
                        out_hbm.at[s, pl.ds(c0 + 8 * i, 8)],
                        sw[b],
                    )

            @pl.when(jnp.logical_not(is_ctx))
            def _():
                pltpu.async_copy(
                    bufs[b], out_hbm.at[s, pl.ds(c0, CHUNK)], sw[b]
                )

        def drain_write(b):
            pltpu.make_async_copy(
                bufs[b], out_hbm.at[0, pl.ds(0, CHUNK)], sw[b]
            ).wait()

        for j in range(4):
            stage_idx(j, j)
        for j in range(3):
            wait_idx(j)
            issue_main(j, j)

        def quad(tt, carry):
            for b in range(NBUF):
                t = tt * NBUF + b
                b3, b4 = (b + 3) % NBUF, (b + 4) % NBUF

                @pl.when(t < n_t)
                def _():
                    wait_main(t, b)
                    issue_write(t, b)

                @pl.when(t + 4 < n_t)
                def _():
                    stage_idx(t + 4, b4)

                @pl.when(t + 3 < n_t)
                def _():
                    wait_idx(b3)

                    @pl.when(t >= 2)
                    def _():
                        drain_write(b3)

                    issue_main(t + 3, b3)
            return carry

        lax.fori_loop(0, ROUNDS // NBUF, quad, 0)
        for b in range(NBUF):
            drain_write(b)

    return sc_kernel


_SC_KERNEL = _make_kernel()


@jax.jit
def kernel(tokenized_prompts, token_embedding, ctx):
    cols = jnp.concatenate(
        [tokenized_prompts[:, :1], tokenized_prompts[:, 1 + N_CTX_K :]], axis=1
    )
    idx = cols.T.reshape(-1)
    ctxb = jnp.broadcast_to(ctx[:, None, :], (N_CTX_K, 8, CTX_DIM_K))
    out_t = _SC_KERNEL(idx, token_embedding, ctxb)
    return jnp.transpose(out_t, (1, 0, 2))
